# Initial kernel scaffold; baseline (speedup 1.0000x reference)
import jax
import jax.numpy as jnp
from jax.experimental import pallas as pl


def kernel(user_emb_pi, h_I_all, user_idx, N, W1, b1, W2, b2, W_agg, b_agg_lin, b_agg):
    raise NotImplementedError("write your pallas kernel here")


# speedup vs baseline: 3.4054x; 3.4054x over previous
import functools

import jax
import jax.numpy as jnp
from jax import lax
from jax.experimental import pallas as pl
from jax.experimental.pallas import tpu as pltpu
from jax.experimental.pallas import tpu_sc as plsc

N_USERS = 100000
D = 64
H = 32
B = 8192
DEG = 64

NC = 2
NS = 16
NW = NC * NS
UPW = B // NW
NSLOT = 4


def _sc_gather_body(uidx_hbm, n_hbm, h_hbm, pi_hbm, hg_out, pig_out,
                    uidx_v, friends_v, pibuf, hbuf, psem, gsem, wsem):
    c = lax.axis_index("c")
    s = lax.axis_index("s")
    wid = s * NC + c
    ubase = wid * UPW

    pltpu.sync_copy(uidx_hbm.at[pl.ds(ubase, UPW)], uidx_v)

    for j in range(UPW // 128):
        idx = uidx_v.at[pl.ds(j * 128, 128)]
        pltpu.async_copy(n_hbm.at[idx], friends_v.at[pl.ds(j * 128, 128)], psem)
        pltpu.async_copy(pi_hbm.at[idx], pibuf.at[pl.ds(j * 128, 128)], psem)
    for j in range(UPW // 128):
        pltpu.make_async_copy(
            n_hbm.at[uidx_v.at[pl.ds(0, 128)]],
            friends_v.at[pl.ds(0, 128)], psem).wait()
        pltpu.make_async_copy(
            pi_hbm.at[uidx_v.at[pl.ds(0, 128)]],
            pibuf.at[pl.ds(0, 128)], psem).wait()
    pltpu.sync_copy(pibuf, pig_out.at[pl.ds(ubase, UPW)])

    def fire_gather(u, slot):
        pltpu.async_copy(h_hbm.at[friends_v.at[u]], hbuf.at[slot],
                         gsem.at[slot])

    def wait_gather(slot):
        pltpu.make_async_copy(h_hbm.at[friends_v.at[0]], hbuf.at[slot],
                              gsem.at[slot]).wait()

    def fire_write(u, slot):
        pltpu.async_copy(hbuf.at[slot],
                         hg_out.at[pl.ds((ubase + u) * DEG, DEG)],
                         wsem.at[slot])

    def wait_write(slot):
        pltpu.make_async_copy(hbuf.at[slot],
                              hg_out.at[pl.ds(0, DEG)], wsem.at[slot]).wait()

    fire_gather(0, 0)
    fire_gather(1, 1)

    def loop(u, _):
        nslot = lax.rem(u + 2, NSLOT)

        @pl.when(u >= 2)
        def _():
            wait_write(nslot)

        fire_gather(u + 2, nslot)
        slot = lax.rem(u, NSLOT)
        wait_gather(slot)
        fire_write(u, slot)
        return 0

    lax.fori_loop(0, UPW - 2, loop, 0)

    for u in (UPW - 2, UPW - 1):
        slot = u % NSLOT
        wait_gather(slot)
        fire_write(u, slot)
    for slot in range(NSLOT):
        wait_write(slot)


@jax.jit
def _sc_gather(user_idx, n_mat, h_all, pi_all):
    mesh = plsc.VectorSubcoreMesh(core_axis_name="c", subcore_axis_name="s")
    return pl.kernel(
        _sc_gather_body,
        out_type=(
            jax.ShapeDtypeStruct((B * DEG, D), jnp.float32),
            jax.ShapeDtypeStruct((B, D), jnp.float32),
        ),
        mesh=mesh,
        compiler_params=pltpu.CompilerParams(use_tc_tiling_on_sc=False),
        scratch_types=[
            pltpu.VMEM((UPW,), jnp.int32),
            pltpu.VMEM((UPW, DEG), jnp.int32),
            pltpu.VMEM((UPW, D), jnp.float32),
            pltpu.VMEM((NSLOT, DEG, D), jnp.float32),
            pltpu.SemaphoreType.DMA,
            pltpu.SemaphoreType.DMA((NSLOT,)),
            pltpu.SemaphoreType.DMA((NSLOT,)),
        ],
    )(user_idx, n_mat, h_all, pi_all)


BBU = 256


def _tc_body(pig_ref, hg_ref, w1_ref, b1_ref, w2_ref, b2_ref, wagg_ref,
             bal_ref, bagg_ref, out_ref):
    x = hg_ref[...]
    w1 = w1_ref[...]
    w1h = w1[:, :D]
    w1p = w1[:, D:]
    xw = lax.dot_general(x, w1h, (((1,), (1,)), ((), ())),
                         preferred_element_type=jnp.float32)
    pi = pig_ref[...]
    piw = lax.dot_general(pi, w1p, (((1,), (1,)), ((), ())),
                          preferred_element_type=jnp.float32)
    hid = jnp.maximum(
        xw.reshape(BBU, DEG, H) + piw[:, None, :] + b1_ref[...][None, None, :],
        0.0)
    beta = jnp.sum(hid * w2_ref[...][0][None, None, :], axis=-1) + b2_ref[0]
    beta = beta - jnp.max(beta, axis=-1, keepdims=True)
    e = jnp.exp(beta)
    beta = e / jnp.sum(e, axis=-1, keepdims=True)
    x3 = x.reshape(BBU, DEG, D)
    ws = jnp.sum(x3 * beta[:, :, None], axis=1)
    out = lax.dot_general(ws, wagg_ref[...], (((1,), (1,)), ((), ())),
                          preferred_element_type=jnp.float32)
    out_ref[...] = jnp.maximum(
        out + bal_ref[...][None, :] + bagg_ref[...][None, :], 0.0)


@jax.jit
def _tc_compute(pig, hg, w1, b1, w2, b2, wagg, bal, bagg):
    nblk = B // BBU
    const = lambda shape: pl.BlockSpec(shape, lambda i: (0,) * len(shape))
    return pl.pallas_call(
        _tc_body,
        grid=(nblk,),
        in_specs=[
            pl.BlockSpec((BBU, D), lambda i: (i, 0)),
            pl.BlockSpec((BBU * DEG, D), lambda i: (i, 0)),
            const((H, 2 * D)),
            const((H,)),
            const((1, H)),
            const((1,)),
            const((D, D)),
            const((D,)),
            const((D,)),
        ],
        out_specs=pl.BlockSpec((BBU, D), lambda i: (i, 0)),
        out_shape=jax.ShapeDtypeStruct((B, D), jnp.float32),
    )(pig, hg, w1, b1, w2, b2, wagg, bal, bagg)


def kernel(user_emb_pi, h_I_all, user_idx, N, W1, b1, W2, b2, W_agg,
           b_agg_lin, b_agg):
    hg, pig = _sc_gather(user_idx, N, h_I_all, user_emb_pi)
    return _tc_compute(pig, hg, W1, b1, W2, b2, W_agg, b_agg_lin, b_agg)


# speedup vs baseline: 3.4604x; 1.0161x over previous
import functools

import jax
import jax.numpy as jnp
from jax import lax
from jax.experimental import pallas as pl
from jax.experimental.pallas import tpu as pltpu
from jax.experimental.pallas import tpu_sc as plsc

N_USERS = 100000
D = 64
H = 32
B = 8192
DEG = 64

NC = 2
NS = 16
NW = NC * NS
NSLOT = 4
NCHUNK = 4
CU = B // NCHUNK


def _sc_gather_body(upw, uidx_hbm, n_hbm, h_hbm, pi_hbm, hg_out, pig_out,
                    uidx_v, friends_v, pibuf, hbuf, psem, gsem, wsem):
    c = lax.axis_index("c")
    s = lax.axis_index("s")
    wid = s * NC + c
    ubase = wid * upw

    pltpu.sync_copy(uidx_hbm.at[pl.ds(ubase, upw)], uidx_v)

    idw = min(upw, 128)
    for j in range(upw // idw):
        idx = uidx_v.at[pl.ds(j * idw, idw)]
        pltpu.async_copy(n_hbm.at[idx], friends_v.at[pl.ds(j * idw, idw)], psem)
        pltpu.async_copy(pi_hbm.at[idx], pibuf.at[pl.ds(j * idw, idw)], psem)
    for j in range(upw // idw):
        pltpu.make_async_copy(
            n_hbm.at[uidx_v.at[pl.ds(0, idw)]],
            friends_v.at[pl.ds(0, idw)], psem).wait()
        pltpu.make_async_copy(
            pi_hbm.at[uidx_v.at[pl.ds(0, idw)]],
            pibuf.at[pl.ds(0, idw)], psem).wait()
    pltpu.sync_copy(pibuf, pig_out.at[pl.ds(ubase, upw)])

    def fire_gather(u, slot):
        pltpu.async_copy(h_hbm.at[friends_v.at[u]], hbuf.at[slot],
                         gsem.at[slot])

    def wait_gather(slot):
        pltpu.make_async_copy(h_hbm.at[friends_v.at[0]], hbuf.at[slot],
                              gsem.at[slot]).wait()

    def fire_write(u, slot):
        pltpu.async_copy(hbuf.at[slot],
                         hg_out.at[pl.ds((ubase + u) * DEG, DEG)],
                         wsem.at[slot])

    def wait_write(slot):
        pltpu.make_async_copy(hbuf.at[slot],
                              hg_out.at[pl.ds(0, DEG)], wsem.at[slot]).wait()

    fire_gather(0, 0)
    fire_gather(1, 1)

    def loop(u, _):
        nslot = lax.rem(u + 2, NSLOT)

        @pl.when(u >= 2)
        def _():
            wait_write(nslot)

        fire_gather(u + 2, nslot)
        slot = lax.rem(u, NSLOT)
        wait_gather(slot)
        fire_write(u, slot)
        return 0

    lax.fori_loop(0, upw - 2, loop, 0)

    for u in (upw - 2, upw - 1):
        slot = u % NSLOT
        wait_gather(slot)
        fire_write(u, slot)
    for slot in range(NSLOT):
        wait_write(slot)


@jax.jit
def _sc_gather(user_idx, n_mat, h_all, pi_all):
    cu = user_idx.shape[0]
    upw = cu // NW
    mesh = plsc.VectorSubcoreMesh(core_axis_name="c", subcore_axis_name="s")
    return pl.kernel(
        functools.partial(_sc_gather_body, upw),
        out_type=(
            jax.ShapeDtypeStruct((cu * DEG, D), jnp.float32),
            jax.ShapeDtypeStruct((cu, D), jnp.float32),
        ),
        mesh=mesh,
        compiler_params=pltpu.CompilerParams(use_tc_tiling_on_sc=False),
        scratch_types=[
            pltpu.VMEM((upw,), jnp.int32),
            pltpu.VMEM((upw, DEG), jnp.int32),
            pltpu.VMEM((upw, D), jnp.float32),
            pltpu.VMEM((NSLOT, DEG, D), jnp.float32),
            pltpu.SemaphoreType.DMA,
            pltpu.SemaphoreType.DMA((NSLOT,)),
            pltpu.SemaphoreType.DMA((NSLOT,)),
        ],
    )(user_idx, n_mat, h_all, pi_all)


BBU = 256


def _tc_body(pig_ref, hg_ref, w1_ref, b1_ref, w2_ref, b2_ref, wagg_ref,
             bal_ref, bagg_ref, out_ref):
    x = hg_ref[...]
    w1 = w1_ref[...]
    w1h = w1[:, :D]
    w1p = w1[:, D:]
    xw = lax.dot_general(x, w1h, (((1,), (1,)), ((), ())),
                         preferred_element_type=jnp.float32)
    pi = pig_ref[...]
    piw = lax.dot_general(pi, w1p, (((1,), (1,)), ((), ())),
                          preferred_element_type=jnp.float32)
    hid = jnp.maximum(
        xw.reshape(BBU, DEG, H) + piw[:, None, :] + b1_ref[...][None, None, :],
        0.0)
    beta = jnp.sum(hid * w2_ref[...][0][None, None, :], axis=-1)
    beta = beta + b2_ref[0]
    beta = beta - jnp.max(beta, axis=-1, keepdims=True)
    e = jnp.exp(beta)
    beta = e / jnp.sum(e, axis=-1, keepdims=True)
    x3 = x.reshape(BBU, DEG, D)
    ws = jnp.sum(x3 * beta[:, :, None], axis=1)
    out = lax.dot_general(ws, wagg_ref[...], (((1,), (1,)), ((), ())),
                          preferred_element_type=jnp.float32)
    out_ref[...] = jnp.maximum(
        out + bal_ref[...][None, :] + bagg_ref[...][None, :], 0.0)


@jax.jit
def _tc_compute(pig, hg, w1, b1, w2, b2, wagg, bal, bagg):
    cu = pig.shape[0]
    nblk = cu // BBU
    const = lambda shape: pl.BlockSpec(shape, lambda i: (0,) * len(shape))
    return pl.pallas_call(
        _tc_body,
        grid=(nblk,),
        in_specs=[
            pl.BlockSpec((BBU, D), lambda i: (i, 0)),
            pl.BlockSpec((BBU * DEG, D), lambda i: (i, 0)),
            const((H, 2 * D)),
            const((H,)),
            const((1, H)),
            const((1,)),
            const((D, D)),
            const((D,)),
            const((D,)),
        ],
        out_specs=pl.BlockSpec((BBU, D), lambda i: (i, 0)),
        out_shape=jax.ShapeDtypeStruct((cu, D), jnp.float32),
    )(pig, hg, w1, b1, w2, b2, wagg, bal, bagg)


def kernel(user_emb_pi, h_I_all, user_idx, N, W1, b1, W2, b2, W_agg,
           b_agg_lin, b_agg):
    outs = []
    for c in range(NCHUNK):
        ui = user_idx[c * CU:(c + 1) * CU]
        hg, pig = _sc_gather(ui, N, h_I_all, user_emb_pi)
        outs.append(_tc_compute(pig, hg, W1, b1, W2, b2, W_agg,
                                b_agg_lin, b_agg))
    return jnp.concatenate(outs, axis=0)


# speedup vs baseline: 3.4616x; 1.0004x over previous
import functools

import jax
import jax.numpy as jnp
from jax import lax
from jax.experimental import pallas as pl
from jax.experimental.pallas import tpu as pltpu
from jax.experimental.pallas import tpu_sc as plsc

N_USERS = 100000
D = 64
H = 32
B = 8192
DEG = 64

NC = 2
NS = 16
NW = NC * NS
NSLOT = 4
NCHUNK = 2
CU = B // NCHUNK


def _sc_gather_body(upw, uidx_hbm, n_hbm, h_hbm, pi_hbm, hg_out, pig_out,
                    uidx_v, friends_v, pibuf, hbuf, psem, gsem, wsem):
    c = lax.axis_index("c")
    s = lax.axis_index("s")
    wid = s * NC + c
    ubase = wid * upw

    pltpu.sync_copy(uidx_hbm.at[pl.ds(ubase, upw)], uidx_v)

    idw = min(upw, 128)
    for j in range(upw // idw):
        idx = uidx_v.at[pl.ds(j * idw, idw)]
        pltpu.async_copy(n_hbm.at[idx], friends_v.at[pl.ds(j * idw, idw)], psem)
        pltpu.async_copy(pi_hbm.at[idx], pibuf.at[pl.ds(j * idw, idw)], psem)
    for j in range(upw // idw):
        pltpu.make_async_copy(
            n_hbm.at[uidx_v.at[pl.ds(0, idw)]],
            friends_v.at[pl.ds(0, idw)], psem).wait()
        pltpu.make_async_copy(
            pi_hbm.at[uidx_v.at[pl.ds(0, idw)]],
            pibuf.at[pl.ds(0, idw)], psem).wait()
    pltpu.sync_copy(pibuf, pig_out.at[pl.ds(ubase, upw)])

    def fire_gather(u, slot):
        pltpu.async_copy(h_hbm.at[friends_v.at[u]], hbuf.at[slot],
                         gsem.at[slot])

    def wait_gather(slot):
        pltpu.make_async_copy(h_hbm.at[friends_v.at[0]], hbuf.at[slot],
                              gsem.at[slot]).wait()

    def fire_write(u, slot):
        pltpu.async_copy(hbuf.at[slot],
                         hg_out.at[pl.ds((ubase + u) * DEG, DEG)],
                         wsem.at[slot])

    def wait_write(slot):
        pltpu.make_async_copy(hbuf.at[slot],
                              hg_out.at[pl.ds(0, DEG)], wsem.at[slot]).wait()

    fire_gather(0, 0)
    fire_gather(1, 1)

    def loop(u, _):
        nslot = lax.rem(u + 2, NSLOT)

        @pl.when(u >= 2)
        def _():
            wait_write(nslot)

        fire_gather(u + 2, nslot)
        slot = lax.rem(u, NSLOT)
        wait_gather(slot)
        fire_write(u, slot)
        return 0

    lax.fori_loop(0, upw - 2, loop, 0)

    for u in (upw - 2, upw - 1):
        slot = u % NSLOT
        wait_gather(slot)
        fire_write(u, slot)
    for slot in range(NSLOT):
        wait_write(slot)


@jax.jit
def _sc_gather(user_idx, n_mat, h_all, pi_all):
    cu = user_idx.shape[0]
    upw = cu // NW
    mesh = plsc.VectorSubcoreMesh(core_axis_name="c", subcore_axis_name="s")
    return pl.kernel(
        functools.partial(_sc_gather_body, upw),
        out_type=(
            jax.ShapeDtypeStruct((cu * DEG, D), jnp.float32),
            jax.ShapeDtypeStruct((cu, D), jnp.float32),
        ),
        mesh=mesh,
        compiler_params=pltpu.CompilerParams(use_tc_tiling_on_sc=False),
        scratch_types=[
            pltpu.VMEM((upw,), jnp.int32),
            pltpu.VMEM((upw, DEG), jnp.int32),
            pltpu.VMEM((upw, D), jnp.float32),
            pltpu.VMEM((NSLOT, DEG, D), jnp.float32),
            pltpu.SemaphoreType.DMA,
            pltpu.SemaphoreType.DMA((NSLOT,)),
            pltpu.SemaphoreType.DMA((NSLOT,)),
        ],
    )(user_idx, n_mat, h_all, pi_all)


BBU = 256


def _tc_body(pig_ref, hg_ref, w1_ref, b1_ref, w2_ref, b2_ref, wagg_ref,
             bal_ref, bagg_ref, out_ref):
    x = hg_ref[...]
    w1 = w1_ref[...]
    w1h = w1[:, :D]
    w1p = w1[:, D:]
    xw = lax.dot_general(x, w1h, (((1,), (1,)), ((), ())),
                         preferred_element_type=jnp.float32)
    pi = pig_ref[...]
    piw = lax.dot_general(pi, w1p, (((1,), (1,)), ((), ())),
                          preferred_element_type=jnp.float32)
    hid = jnp.maximum(
        xw.reshape(BBU, DEG, H) + piw[:, None, :] + b1_ref[...][None, None, :],
        0.0)
    beta = jnp.sum(hid * w2_ref[...][0][None, None, :], axis=-1)
    beta = beta + b2_ref[0]
    beta = beta - jnp.max(beta, axis=-1, keepdims=True)
    e = jnp.exp(beta)
    beta = e / jnp.sum(e, axis=-1, keepdims=True)
    x3 = x.reshape(BBU, DEG, D)
    ws = jnp.sum(x3 * beta[:, :, None], axis=1)
    out = lax.dot_general(ws, wagg_ref[...], (((1,), (1,)), ((), ())),
                          preferred_element_type=jnp.float32)
    out_ref[...] = jnp.maximum(
        out + bal_ref[...][None, :] + bagg_ref[...][None, :], 0.0)


@jax.jit
def _tc_compute(pig, hg, w1, b1, w2, b2, wagg, bal, bagg):
    cu = pig.shape[0]
    nblk = cu // BBU
    const = lambda shape: pl.BlockSpec(shape, lambda i: (0,) * len(shape))
    return pl.pallas_call(
        _tc_body,
        grid=(nblk,),
        in_specs=[
            pl.BlockSpec((BBU, D), lambda i: (i, 0)),
            pl.BlockSpec((BBU * DEG, D), lambda i: (i, 0)),
            const((H, 2 * D)),
            const((H,)),
            const((1, H)),
            const((1,)),
            const((D, D)),
            const((D,)),
            const((D,)),
        ],
        out_specs=pl.BlockSpec((BBU, D), lambda i: (i, 0)),
        out_shape=jax.ShapeDtypeStruct((cu, D), jnp.float32),
    )(pig, hg, w1, b1, w2, b2, wagg, bal, bagg)


def kernel(user_emb_pi, h_I_all, user_idx, N, W1, b1, W2, b2, W_agg,
           b_agg_lin, b_agg):
    outs = []
    for c in range(NCHUNK):
        ui = user_idx[c * CU:(c + 1) * CU]
        hg, pig = _sc_gather(ui, N, h_I_all, user_emb_pi)
        outs.append(_tc_compute(pig, hg, W1, b1, W2, b2, W_agg,
                                b_agg_lin, b_agg))
    return jnp.concatenate(outs, axis=0)
